# baseline (speedup 1.0000x reference)
import jax
import jax.numpy as jnp
from jax import lax
from jax.experimental import pallas as pl
from jax.experimental.pallas import tpu as pltpu

PE = 8192
B = 4096
BR = 64


def _body(pos_ref, out_ref):
    p = pos_ref[0, 0, :]
    p = jnp.minimum(jnp.maximum(p, 0), PE - 1)
    col = lax.broadcasted_iota(jnp.int32, (BR, PE), 1)
    out_ref[...] = (col == p[:, None]).astype(jnp.float32)


@jax.jit
def _one_hot(position):
    return pl.pallas_call(
        _body,
        grid=(B // BR,),
        in_specs=[pl.BlockSpec((1, 1, BR), lambda i: (i, 0, 0))],
        out_specs=pl.BlockSpec((BR, PE), lambda i: (i, 0)),
        out_shape=jax.ShapeDtypeStruct((B, PE), jnp.float32),
    )(position.reshape(B // BR, 1, BR))


def kernel(position):
    if position.ndim > 1 and position.shape[-1] == 1:
        position = jnp.squeeze(position, axis=-1)
    return _one_hot(position.astype(jnp.int32))


# speedup vs baseline: 1.3330x; 1.3330x over previous
import jax
import jax.numpy as jnp
from jax import lax
from jax.experimental import pallas as pl
from jax.experimental.pallas import tpu as pltpu

PE = 8192
B = 4096
BR = 128


def _body(pos_ref, out_ref):
    i = pl.program_id(0)
    p = pos_ref[pl.ds(i * BR, BR)]
    p = jnp.minimum(jnp.maximum(p, 0), PE - 1)
    col = lax.broadcasted_iota(jnp.int32, (BR, PE), 1)
    out_ref[...] = (col == p[:, None]).astype(jnp.float32)


@jax.jit
def _one_hot(position):
    return pl.pallas_call(
        _body,
        grid=(B // BR,),
        in_specs=[pl.BlockSpec((B,), lambda i: (0,))],
        out_specs=pl.BlockSpec((BR, PE), lambda i: (i, 0)),
        out_shape=jax.ShapeDtypeStruct((B, PE), jnp.float32),
    )(position)


def kernel(position):
    if position.ndim > 1 and position.shape[-1] == 1:
        position = jnp.squeeze(position, axis=-1)
    return _one_hot(position.astype(jnp.int32))


# speedup vs baseline: 1.3406x; 1.0057x over previous
import jax
import jax.numpy as jnp
from jax import lax
from jax.experimental import pallas as pl
from jax.experimental.pallas import tpu as pltpu

PE = 8192
B = 4096
BR = 128


def _body(pos_ref, out_ref):
    p = jnp.minimum(jnp.maximum(pos_ref[...], 0), PE - 1)
    col = lax.broadcasted_iota(jnp.int32, (BR, PE), 1)
    out_ref[...] = (col == p[:, None]).astype(jnp.float32)


@jax.jit
def _one_hot(position):
    return pl.pallas_call(
        _body,
        grid=(B // BR,),
        in_specs=[pl.BlockSpec((BR,), lambda i: (i,))],
        out_specs=pl.BlockSpec((BR, PE), lambda i: (i, 0)),
        out_shape=jax.ShapeDtypeStruct((B, PE), jnp.float32),
    )(position)


def kernel(position):
    if position.ndim > 1 and position.shape[-1] == 1:
        position = jnp.squeeze(position, axis=-1)
    return _one_hot(position.astype(jnp.int32))
